# Initial kernel scaffold; baseline (speedup 1.0000x reference)
import jax
import jax.numpy as jnp
from jax.experimental import pallas as pl


def kernel(x, edge_index, W1_l, b1, W1_r, W2_l, b2, W2_r):
    raise NotImplementedError("write your pallas kernel here")


# speedup vs baseline: 3.3039x; 3.3039x over previous
import functools

import jax
import jax.numpy as jnp
from jax import lax
from jax.experimental import pallas as pl
from jax.experimental.pallas import tpu as pltpu
from jax.experimental.pallas import tpu_sc as plsc

N = 10000
E = 320000
D_IN = 128
D_HID = 256

NC = 2
NS = 16
B = 128

NPAD = 10008
E_PAD = 327680
ROWS_L1 = 80
ROWS_L2 = 160
ZCH = 624
NDEG = 10240
DCH = 640
ICH = 16

_mesh = plsc.VectorSubcoreMesh(
    core_axis_name="c", subcore_axis_name="s", num_cores=NC, num_subcores=NS
)


def _zero_acc(zrows, acc, s):
    pltpu.sync_copy(zrows.at[pl.ds(s * ZCH, ZCH)], acc.at[pl.ds(s * ZCH, ZCH)])

    @pl.when(s == NS - 1)
    def _():
        pltpu.sync_copy(zrows.at[pl.ds(NS * ZCH, NPAD - NS * ZCH)],
                        acc.at[pl.ds(NS * ZCH, NPAD - NS * ZCH)])


def _copy_out_rows(acc, out_hbm, c, s):
    pltpu.sync_copy(acc.at[pl.ds(s * ZCH, ZCH)], out_hbm.at[c, pl.ds(s * ZCH, ZCH)])

    @pl.when(s == NS - 1)
    def _():
        pltpu.sync_copy(acc.at[pl.ds(NS * ZCH, N - NS * ZCH)],
                        out_hbm.at[c, pl.ds(NS * ZCH, N - NS * ZCH)])


@functools.partial(
    pl.kernel,
    out_type=(
        jax.ShapeDtypeStruct((NC, N, D_IN), jnp.float32),
        jax.ShapeDtypeStruct((NDEG,), jnp.float32),
        jax.ShapeDtypeStruct((NDEG,), jnp.float32),
    ),
    mesh=_mesh,
    scratch_types=[
        pltpu.VMEM_SHARED((NPAD, D_IN), jnp.float32),
        pltpu.VMEM_SHARED((NDEG,), jnp.float32),
        pltpu.VMEM((ICH, B), jnp.int32),
        pltpu.VMEM((ICH, B), jnp.int32),
        pltpu.VMEM((B, D_IN), jnp.float32),
        pltpu.VMEM((B,), jnp.float32),
        pltpu.SemaphoreType.DMA,
    ],
)
def _sc_layer1(x_hbm, src_hbm, dst_hbm, zrows, zdeg,
               psum, pdeg0, pdeg1, acc, dacc, idx_s, idx_d, rows, ones_v, sem):
    c = lax.axis_index("c")
    s = lax.axis_index("s")

    _zero_acc(zrows, acc, s)
    pltpu.sync_copy(zdeg.at[pl.ds(s * DCH, DCH)], dacc.at[pl.ds(s * DCH, DCH)])
    for i in range(B // 16):
        ones_v[pl.ds(16 * i, 16)] = jnp.ones((16,), jnp.float32)

    rb = (c * NS + s) * ROWS_L1
    plsc.subcore_barrier()

    def chunk(k, carry):
        pltpu.sync_copy(src_hbm.at[pl.ds(rb + k * ICH, ICH)], idx_s)
        pltpu.sync_copy(dst_hbm.at[pl.ds(rb + k * ICH, ICH)], idx_d)

        def body(j, carry2):
            pltpu.async_copy(x_hbm.at[idx_s.at[j]], rows, sem).wait()
            pltpu.sync_copy(rows, acc.at[idx_d.at[j]], add=True)
            pltpu.sync_copy(ones_v, dacc.at[idx_d.at[j]], add=True)
            return carry2

        return lax.fori_loop(0, ICH, body, carry)

    lax.fori_loop(0, ROWS_L1 // ICH, chunk, 0)

    plsc.subcore_barrier()
    _copy_out_rows(acc, psum, c, s)

    @pl.when(c == 0)
    def _():
        pltpu.sync_copy(dacc.at[pl.ds(s * DCH, DCH)], pdeg0.at[pl.ds(s * DCH, DCH)])

    @pl.when(c == 1)
    def _():
        pltpu.sync_copy(dacc.at[pl.ds(s * DCH, DCH)], pdeg1.at[pl.ds(s * DCH, DCH)])


@functools.partial(
    pl.kernel,
    out_type=jax.ShapeDtypeStruct((NC, N, D_HID // 2), jnp.float32),
    mesh=_mesh,
    scratch_types=[
        pltpu.VMEM_SHARED((NPAD, D_HID // 2), jnp.float32),
        pltpu.VMEM((ICH, B), jnp.int32),
        pltpu.VMEM((ICH, B), jnp.int32),
        pltpu.VMEM((B, D_HID // 2), jnp.float32),
        pltpu.SemaphoreType.DMA,
    ],
)
def _sc_layer2(h1a_hbm, h1b_hbm, src_hbm, dst_hbm, zrows,
               psum, acc, idx_s, idx_d, rows, sem):
    c = lax.axis_index("c")
    s = lax.axis_index("s")

    _zero_acc(zrows, acc, s)

    rb = s * ROWS_L2
    plsc.subcore_barrier()

    def make_chunk(h_hbm):
        def chunk(k, carry):
            pltpu.sync_copy(src_hbm.at[pl.ds(rb + k * ICH, ICH)], idx_s)
            pltpu.sync_copy(dst_hbm.at[pl.ds(rb + k * ICH, ICH)], idx_d)

            def body(j, carry2):
                pltpu.async_copy(h_hbm.at[idx_s.at[j]], rows, sem).wait()
                pltpu.sync_copy(rows, acc.at[idx_d.at[j]], add=True)
                return carry2

            return lax.fori_loop(0, ICH, body, carry)

        return chunk

    @pl.when(c == 0)
    def _():
        lax.fori_loop(0, ROWS_L2 // ICH, make_chunk(h1a_hbm), 0)

    @pl.when(c == 1)
    def _():
        lax.fori_loop(0, ROWS_L2 // ICH, make_chunk(h1b_hbm), 0)

    plsc.subcore_barrier()
    _copy_out_rows(acc, psum, c, s)


def _tc1_body(psum_ref, pd0_ref, pd1_ref, x_ref, wl_ref, wr_ref, b_ref,
              h1a_ref, h1b_ref, inv_ref):
    deg = pd0_ref[...] + pd1_ref[...]
    inv = 1.0 / jnp.maximum(deg, 1.0)
    aggr = (psum_ref[0] + psum_ref[1]) * inv
    h1 = (jnp.dot(aggr, wl_ref[...], preferred_element_type=jnp.float32)
          + jnp.dot(x_ref[...], wr_ref[...], preferred_element_type=jnp.float32)
          + b_ref[...])
    h1a_ref[...] = h1[:, :D_IN]
    h1b_ref[...] = h1[:, D_IN:]
    inv_ref[...] = inv


def _tc2_body(s2_ref, inv_ref, h1a_ref, h1b_ref,
              wla_ref, wlb_ref, wra_ref, wrb_ref, b_ref, out_ref):
    inv = inv_ref[...]
    o = (jnp.dot(s2_ref[0] * inv, wla_ref[...], preferred_element_type=jnp.float32)
         + jnp.dot(s2_ref[1] * inv, wlb_ref[...], preferred_element_type=jnp.float32)
         + jnp.dot(h1a_ref[...], wra_ref[...], preferred_element_type=jnp.float32)
         + jnp.dot(h1b_ref[...], wrb_ref[...], preferred_element_type=jnp.float32)
         + b_ref[...])
    out_ref[...] = jnp.maximum(o, 0.0)


_R = 2000


def _tc_layer1(psum, pd0, pd1, x, w1lT, w1rT, b1r):
    H = D_HID // 2
    return pl.pallas_call(
        _tc1_body,
        grid=(N // _R,),
        in_specs=[
            pl.BlockSpec((NC, _R, D_IN), lambda i: (0, i, 0)),
            pl.BlockSpec((_R, 1), lambda i: (i, 0)),
            pl.BlockSpec((_R, 1), lambda i: (i, 0)),
            pl.BlockSpec((_R, D_IN), lambda i: (i, 0)),
            pl.BlockSpec((D_IN, D_HID), lambda i: (0, 0)),
            pl.BlockSpec((D_IN, D_HID), lambda i: (0, 0)),
            pl.BlockSpec((1, D_HID), lambda i: (0, 0)),
        ],
        out_specs=[
            pl.BlockSpec((_R, H), lambda i: (i, 0)),
            pl.BlockSpec((_R, H), lambda i: (i, 0)),
            pl.BlockSpec((_R, 1), lambda i: (i, 0)),
        ],
        out_shape=[
            jax.ShapeDtypeStruct((N, H), jnp.float32),
            jax.ShapeDtypeStruct((N, H), jnp.float32),
            jax.ShapeDtypeStruct((N, 1), jnp.float32),
        ],
    )(psum, pd0, pd1, x, w1lT, w1rT, b1r)


def _tc_layer2(psum2, inv, h1a, h1b, w2la, w2lb, w2ra, w2rb, b2r):
    H = D_HID // 2
    return pl.pallas_call(
        _tc2_body,
        grid=(N // _R,),
        in_specs=[
            pl.BlockSpec((NC, _R, H), lambda i: (0, i, 0)),
            pl.BlockSpec((_R, 1), lambda i: (i, 0)),
            pl.BlockSpec((_R, H), lambda i: (i, 0)),
            pl.BlockSpec((_R, H), lambda i: (i, 0)),
            pl.BlockSpec((H, D_HID), lambda i: (0, 0)),
            pl.BlockSpec((H, D_HID), lambda i: (0, 0)),
            pl.BlockSpec((H, D_HID), lambda i: (0, 0)),
            pl.BlockSpec((H, D_HID), lambda i: (0, 0)),
            pl.BlockSpec((1, D_HID), lambda i: (0, 0)),
        ],
        out_specs=pl.BlockSpec((_R, D_HID), lambda i: (i, 0)),
        out_shape=jax.ShapeDtypeStruct((N, D_HID), jnp.float32),
    )(psum2, inv, h1a, h1b, w2la, w2lb, w2ra, w2rb, b2r)


def kernel(x, edge_index, W1_l, b1, W1_r, W2_l, b2, W2_r):
    ei = edge_index.astype(jnp.int32)
    pad = E_PAD - E
    src2d = jnp.concatenate([ei[0], jnp.zeros((pad,), jnp.int32)]).reshape(E_PAD // B, B)
    dst2d = jnp.concatenate([ei[1], jnp.full((pad,), N, jnp.int32)]).reshape(E_PAD // B, B)
    zrows = jnp.zeros((NPAD, D_IN), jnp.float32)
    zdeg = jnp.zeros((NDEG,), jnp.float32)

    psum, pdeg0, pdeg1 = _sc_layer1(x, src2d, dst2d, zrows, zdeg)
    h1a, h1b, inv = _tc_layer1(
        psum, pdeg0[:N].reshape(N, 1), pdeg1[:N].reshape(N, 1), x,
        W1_l.T, W1_r.T, b1.reshape(1, D_HID))
    psum2 = _sc_layer2(h1a, h1b, src2d, dst2d, zrows)
    out = _tc_layer2(
        psum2, inv, h1a, h1b,
        W2_l.T[:D_IN], W2_l.T[D_IN:], W2_r.T[:D_IN], W2_r.T[D_IN:],
        b2.reshape(1, D_HID))
    return out


# speedup vs baseline: 3.9250x; 1.1880x over previous
import functools

import jax
import jax.numpy as jnp
from jax import lax
from jax.experimental import pallas as pl
from jax.experimental.pallas import tpu as pltpu
from jax.experimental.pallas import tpu_sc as plsc

N = 10000
E = 320000
D_IN = 128
D_HID = 256

NC = 2
NS = 16
B = 128

NPAD = 10008
E_PAD = 327680
ROWS_L1 = 80
ROWS_L2 = 160
ZCH = 624
NDEG = 10240
DCH = 640
ICH = 40

_mesh = plsc.VectorSubcoreMesh(
    core_axis_name="c", subcore_axis_name="s", num_cores=NC, num_subcores=NS
)


def _zero_acc(zrows, acc, s):
    pltpu.sync_copy(zrows.at[pl.ds(s * ZCH, ZCH)], acc.at[pl.ds(s * ZCH, ZCH)])

    @pl.when(s == NS - 1)
    def _():
        pltpu.sync_copy(zrows.at[pl.ds(NS * ZCH, NPAD - NS * ZCH)],
                        acc.at[pl.ds(NS * ZCH, NPAD - NS * ZCH)])


def _copy_out_rows(acc, out_hbm, c, s):
    pltpu.sync_copy(acc.at[pl.ds(s * ZCH, ZCH)], out_hbm.at[c, pl.ds(s * ZCH, ZCH)])

    @pl.when(s == NS - 1)
    def _():
        pltpu.sync_copy(acc.at[pl.ds(NS * ZCH, N - NS * ZCH)],
                        out_hbm.at[c, pl.ds(NS * ZCH, N - NS * ZCH)])


@functools.partial(
    pl.kernel,
    out_type=(
        jax.ShapeDtypeStruct((NC, N, D_IN), jnp.float32),
        jax.ShapeDtypeStruct((NDEG,), jnp.float32),
        jax.ShapeDtypeStruct((NDEG,), jnp.float32),
    ),
    mesh=_mesh,
    scratch_types=[
        pltpu.VMEM_SHARED((NPAD, D_IN), jnp.float32),
        pltpu.VMEM_SHARED((NDEG,), jnp.float32),
        pltpu.VMEM((ICH, B), jnp.int32),
        pltpu.VMEM((ICH, B), jnp.int32),
        pltpu.VMEM((B, D_IN), jnp.float32),
        pltpu.VMEM((B, D_IN), jnp.float32),
        pltpu.VMEM((B,), jnp.float32),
        pltpu.SemaphoreType.DMA,
        pltpu.SemaphoreType.DMA,
    ],
)
def _sc_layer1(x_hbm, src_hbm, dst_hbm, zrows, zdeg,
               psum, pdeg0, pdeg1, acc, dacc, idx_s, idx_d, rows0, rows1,
               ones_v, sem0, sem1):
    c = lax.axis_index("c")
    s = lax.axis_index("s")

    _zero_acc(zrows, acc, s)
    pltpu.sync_copy(zdeg.at[pl.ds(s * DCH, DCH)], dacc.at[pl.ds(s * DCH, DCH)])
    for i in range(B // 16):
        ones_v[pl.ds(16 * i, 16)] = jnp.ones((16,), jnp.float32)

    rb = (c * NS + s) * ROWS_L1
    plsc.subcore_barrier()

    def chunk(k, carry):
        pltpu.sync_copy(src_hbm.at[pl.ds(rb + k * ICH, ICH)], idx_s)
        pltpu.sync_copy(dst_hbm.at[pl.ds(rb + k * ICH, ICH)], idx_d)
        pltpu.async_copy(x_hbm.at[idx_s.at[0]], rows0, sem0)

        def pair(p, carry2):
            j = 2 * p
            pltpu.async_copy(x_hbm.at[idx_s.at[j + 1]], rows1, sem1)
            pltpu.make_async_copy(x_hbm.at[idx_s.at[j]], rows0, sem0).wait()
            pltpu.sync_copy(rows0, acc.at[idx_d.at[j]], add=True)
            pltpu.sync_copy(ones_v, dacc.at[idx_d.at[j]], add=True)

            @pl.when(j + 2 < ICH)
            def _():
                pltpu.async_copy(x_hbm.at[idx_s.at[j + 2]], rows0, sem0)

            pltpu.make_async_copy(x_hbm.at[idx_s.at[j + 1]], rows1, sem1).wait()
            pltpu.sync_copy(rows1, acc.at[idx_d.at[j + 1]], add=True)
            pltpu.sync_copy(ones_v, dacc.at[idx_d.at[j + 1]], add=True)
            return carry2

        return lax.fori_loop(0, ICH // 2, pair, carry)

    lax.fori_loop(0, ROWS_L1 // ICH, chunk, 0)

    plsc.subcore_barrier()
    _copy_out_rows(acc, psum, c, s)

    @pl.when(c == 0)
    def _():
        pltpu.sync_copy(dacc.at[pl.ds(s * DCH, DCH)], pdeg0.at[pl.ds(s * DCH, DCH)])

    @pl.when(c == 1)
    def _():
        pltpu.sync_copy(dacc.at[pl.ds(s * DCH, DCH)], pdeg1.at[pl.ds(s * DCH, DCH)])


@functools.partial(
    pl.kernel,
    out_type=jax.ShapeDtypeStruct((NC, N, D_HID // 2), jnp.float32),
    mesh=_mesh,
    scratch_types=[
        pltpu.VMEM_SHARED((NPAD, D_HID // 2), jnp.float32),
        pltpu.VMEM((ICH, B), jnp.int32),
        pltpu.VMEM((ICH, B), jnp.int32),
        pltpu.VMEM((B, D_HID // 2), jnp.float32),
        pltpu.VMEM((B, D_HID // 2), jnp.float32),
        pltpu.SemaphoreType.DMA,
        pltpu.SemaphoreType.DMA,
    ],
)
def _sc_layer2(h1a_hbm, h1b_hbm, src_hbm, dst_hbm, zrows,
               psum, acc, idx_s, idx_d, rows0, rows1, sem0, sem1):
    c = lax.axis_index("c")
    s = lax.axis_index("s")

    _zero_acc(zrows, acc, s)

    rb = s * ROWS_L2
    plsc.subcore_barrier()

    def make_chunk(h_hbm):
        def chunk(k, carry):
            pltpu.sync_copy(src_hbm.at[pl.ds(rb + k * ICH, ICH)], idx_s)
            pltpu.sync_copy(dst_hbm.at[pl.ds(rb + k * ICH, ICH)], idx_d)
            pltpu.async_copy(h_hbm.at[idx_s.at[0]], rows0, sem0)

            def pair(p, carry2):
                j = 2 * p
                pltpu.async_copy(h_hbm.at[idx_s.at[j + 1]], rows1, sem1)
                pltpu.make_async_copy(h_hbm.at[idx_s.at[j]], rows0, sem0).wait()
                pltpu.sync_copy(rows0, acc.at[idx_d.at[j]], add=True)

                @pl.when(j + 2 < ICH)
                def _():
                    pltpu.async_copy(h_hbm.at[idx_s.at[j + 2]], rows0, sem0)

                pltpu.make_async_copy(h_hbm.at[idx_s.at[j + 1]], rows1, sem1).wait()
                pltpu.sync_copy(rows1, acc.at[idx_d.at[j + 1]], add=True)
                return carry2

            return lax.fori_loop(0, ICH // 2, pair, carry)

        return chunk

    @pl.when(c == 0)
    def _():
        lax.fori_loop(0, ROWS_L2 // ICH, make_chunk(h1a_hbm), 0)

    @pl.when(c == 1)
    def _():
        lax.fori_loop(0, ROWS_L2 // ICH, make_chunk(h1b_hbm), 0)

    plsc.subcore_barrier()
    _copy_out_rows(acc, psum, c, s)


def _tc1_body(psum_ref, pd0_ref, pd1_ref, x_ref, wl_ref, wr_ref, b_ref,
              h1a_ref, h1b_ref, inv_ref):
    deg = pd0_ref[...] + pd1_ref[...]
    inv = 1.0 / jnp.maximum(deg, 1.0)
    aggr = (psum_ref[0] + psum_ref[1]) * inv
    h1 = (jnp.dot(aggr, wl_ref[...], preferred_element_type=jnp.float32)
          + jnp.dot(x_ref[...], wr_ref[...], preferred_element_type=jnp.float32)
          + b_ref[...])
    h1a_ref[...] = h1[:, :D_IN]
    h1b_ref[...] = h1[:, D_IN:]
    inv_ref[...] = inv


def _tc2_body(s2_ref, inv_ref, h1a_ref, h1b_ref,
              wla_ref, wlb_ref, wra_ref, wrb_ref, b_ref, out_ref):
    inv = inv_ref[...]
    o = (jnp.dot(s2_ref[0] * inv, wla_ref[...], preferred_element_type=jnp.float32)
         + jnp.dot(s2_ref[1] * inv, wlb_ref[...], preferred_element_type=jnp.float32)
         + jnp.dot(h1a_ref[...], wra_ref[...], preferred_element_type=jnp.float32)
         + jnp.dot(h1b_ref[...], wrb_ref[...], preferred_element_type=jnp.float32)
         + b_ref[...])
    out_ref[...] = jnp.maximum(o, 0.0)


_R = 2000


def _tc_layer1(psum, pd0, pd1, x, w1lT, w1rT, b1r):
    H = D_HID // 2
    return pl.pallas_call(
        _tc1_body,
        grid=(N // _R,),
        in_specs=[
            pl.BlockSpec((NC, _R, D_IN), lambda i: (0, i, 0)),
            pl.BlockSpec((_R, 1), lambda i: (i, 0)),
            pl.BlockSpec((_R, 1), lambda i: (i, 0)),
            pl.BlockSpec((_R, D_IN), lambda i: (i, 0)),
            pl.BlockSpec((D_IN, D_HID), lambda i: (0, 0)),
            pl.BlockSpec((D_IN, D_HID), lambda i: (0, 0)),
            pl.BlockSpec((1, D_HID), lambda i: (0, 0)),
        ],
        out_specs=[
            pl.BlockSpec((_R, H), lambda i: (i, 0)),
            pl.BlockSpec((_R, H), lambda i: (i, 0)),
            pl.BlockSpec((_R, 1), lambda i: (i, 0)),
        ],
        out_shape=[
            jax.ShapeDtypeStruct((N, H), jnp.float32),
            jax.ShapeDtypeStruct((N, H), jnp.float32),
            jax.ShapeDtypeStruct((N, 1), jnp.float32),
        ],
    )(psum, pd0, pd1, x, w1lT, w1rT, b1r)


def _tc_layer2(psum2, inv, h1a, h1b, w2la, w2lb, w2ra, w2rb, b2r):
    H = D_HID // 2
    return pl.pallas_call(
        _tc2_body,
        grid=(N // _R,),
        in_specs=[
            pl.BlockSpec((NC, _R, H), lambda i: (0, i, 0)),
            pl.BlockSpec((_R, 1), lambda i: (i, 0)),
            pl.BlockSpec((_R, H), lambda i: (i, 0)),
            pl.BlockSpec((_R, H), lambda i: (i, 0)),
            pl.BlockSpec((H, D_HID), lambda i: (0, 0)),
            pl.BlockSpec((H, D_HID), lambda i: (0, 0)),
            pl.BlockSpec((H, D_HID), lambda i: (0, 0)),
            pl.BlockSpec((H, D_HID), lambda i: (0, 0)),
            pl.BlockSpec((1, D_HID), lambda i: (0, 0)),
        ],
        out_specs=pl.BlockSpec((_R, D_HID), lambda i: (i, 0)),
        out_shape=jax.ShapeDtypeStruct((N, D_HID), jnp.float32),
    )(psum2, inv, h1a, h1b, w2la, w2lb, w2ra, w2rb, b2r)


def kernel(x, edge_index, W1_l, b1, W1_r, W2_l, b2, W2_r):
    ei = edge_index.astype(jnp.int32)
    pad = E_PAD - E
    src2d = jnp.concatenate([ei[0], jnp.zeros((pad,), jnp.int32)]).reshape(E_PAD // B, B)
    dst2d = jnp.concatenate([ei[1], jnp.full((pad,), N, jnp.int32)]).reshape(E_PAD // B, B)
    zrows = jnp.zeros((NPAD, D_IN), jnp.float32)
    zdeg = jnp.zeros((NDEG,), jnp.float32)

    psum, pdeg0, pdeg1 = _sc_layer1(x, src2d, dst2d, zrows, zdeg)
    h1a, h1b, inv = _tc_layer1(
        psum, pdeg0[:N].reshape(N, 1), pdeg1[:N].reshape(N, 1), x,
        W1_l.T, W1_r.T, b1.reshape(1, D_HID))
    psum2 = _sc_layer2(h1a, h1b, src2d, dst2d, zrows)
    out = _tc_layer2(
        psum2, inv, h1a, h1b,
        W2_l.T[:D_IN], W2_l.T[D_IN:], W2_r.T[:D_IN], W2_r.T[D_IN:],
        b2.reshape(1, D_HID))
    return out


# speedup vs baseline: 3.9274x; 1.0006x over previous
import functools

import numpy as np

import jax
import jax.numpy as jnp
from jax import lax
from jax.experimental import pallas as pl
from jax.experimental.pallas import tpu as pltpu
from jax.experimental.pallas import tpu_sc as plsc

N = 10000
E = 320000
D_IN = 128
D_HID = 256

NC = 2
NS = 16
B = 128

NPAD = 10008
E_PAD = 327680
ROWS_L1 = 80
ROWS_L2 = 160
ZCH = 624
NDEG = 10240
DCH = 640
ICH = 40

_mesh = plsc.VectorSubcoreMesh(
    core_axis_name="c", subcore_axis_name="s", num_cores=NC, num_subcores=NS
)


def _zero_acc(zrows, acc, s):
    pltpu.sync_copy(zrows.at[pl.ds(s * ZCH, ZCH)], acc.at[pl.ds(s * ZCH, ZCH)])

    @pl.when(s == NS - 1)
    def _():
        pltpu.sync_copy(zrows.at[pl.ds(NS * ZCH, NPAD - NS * ZCH)],
                        acc.at[pl.ds(NS * ZCH, NPAD - NS * ZCH)])


def _copy_out_rows(acc, out_hbm, c, s):
    pltpu.sync_copy(acc.at[pl.ds(s * ZCH, ZCH)], out_hbm.at[c, pl.ds(s * ZCH, ZCH)])

    @pl.when(s == NS - 1)
    def _():
        pltpu.sync_copy(acc.at[pl.ds(NS * ZCH, N - NS * ZCH)],
                        out_hbm.at[c, pl.ds(NS * ZCH, N - NS * ZCH)])


@functools.partial(
    pl.kernel,
    out_type=(
        jax.ShapeDtypeStruct((NC, N, D_IN), jnp.float32),
        jax.ShapeDtypeStruct((NDEG,), jnp.float32),
        jax.ShapeDtypeStruct((NDEG,), jnp.float32),
    ),
    mesh=_mesh,
    scratch_types=[
        pltpu.VMEM_SHARED((NPAD, D_IN), jnp.float32),
        pltpu.VMEM_SHARED((NDEG,), jnp.float32),
        pltpu.VMEM((ICH, B), jnp.int32),
        pltpu.VMEM((ICH, B), jnp.int32),
        pltpu.VMEM((B, D_IN), jnp.float32),
        pltpu.VMEM((B, D_IN), jnp.float32),
        pltpu.VMEM((B,), jnp.float32),
        pltpu.SemaphoreType.DMA,
        pltpu.SemaphoreType.DMA,
        pltpu.SemaphoreType.DMA,
    ],
)
def _sc_layer1(x_hbm, src_hbm, dst_hbm, zrows, zdeg,
               psum, pdeg0, pdeg1, acc, dacc, idx_s, idx_d, rows0, rows1,
               ones_v, sem_g0, sem_g1, sem_d):
    c = lax.axis_index("c")
    s = lax.axis_index("s")

    _zero_acc(zrows, acc, s)
    pltpu.sync_copy(zdeg.at[pl.ds(s * DCH, DCH)], dacc.at[pl.ds(s * DCH, DCH)])
    for i in range(B // 16):
        ones_v[pl.ds(16 * i, 16)] = jnp.ones((16,), jnp.float32)

    rb = (c * NS + s) * ROWS_L1
    plsc.subcore_barrier()

    def chunk(k, carry):
        pltpu.sync_copy(src_hbm.at[pl.ds(rb + k * ICH, ICH)], idx_s)
        pltpu.sync_copy(dst_hbm.at[pl.ds(rb + k * ICH, ICH)], idx_d)
        pltpu.async_copy(x_hbm.at[idx_s.at[0]], rows0, sem_g0)

        def pair(p, carry2):
            j = 2 * p
            pltpu.async_copy(x_hbm.at[idx_s.at[j + 1]], rows1, sem_g1)
            pltpu.make_async_copy(x_hbm.at[idx_s.at[0]], rows0, sem_g0).wait()
            pltpu.sync_copy(rows0, acc.at[idx_d.at[j]], add=True)
            pltpu.async_copy(ones_v, dacc.at[idx_d.at[j]], sem_d, add=True)

            @pl.when(j + 2 < ICH)
            def _():
                pltpu.async_copy(x_hbm.at[idx_s.at[j + 2]], rows0, sem_g0)

            pltpu.make_async_copy(x_hbm.at[idx_s.at[0]], rows1, sem_g1).wait()
            pltpu.sync_copy(rows1, acc.at[idx_d.at[j + 1]], add=True)
            pltpu.async_copy(ones_v, dacc.at[idx_d.at[j + 1]], sem_d, add=True)
            return carry2

        carry = lax.fori_loop(0, ICH // 2, pair, carry)

        def drain(j, carry3):
            pltpu.make_async_copy(ones_v, dacc.at[idx_d.at[0]], sem_d).wait()
            return carry3

        return lax.fori_loop(0, ICH, drain, carry)

    lax.fori_loop(0, ROWS_L1 // ICH, chunk, 0)

    plsc.subcore_barrier()
    _copy_out_rows(acc, psum, c, s)

    @pl.when(c == 0)
    def _():
        pltpu.sync_copy(dacc.at[pl.ds(s * DCH, DCH)], pdeg0.at[pl.ds(s * DCH, DCH)])

    @pl.when(c == 1)
    def _():
        pltpu.sync_copy(dacc.at[pl.ds(s * DCH, DCH)], pdeg1.at[pl.ds(s * DCH, DCH)])


@functools.partial(
    pl.kernel,
    out_type=jax.ShapeDtypeStruct((NC, N, D_HID // 2), jnp.float32),
    mesh=_mesh,
    scratch_types=[
        pltpu.VMEM_SHARED((NPAD, D_HID // 2), jnp.float32),
        pltpu.VMEM((ICH, B), jnp.int32),
        pltpu.VMEM((ICH, B), jnp.int32),
        pltpu.VMEM((B, D_HID // 2), jnp.float32),
        pltpu.VMEM((B, D_HID // 2), jnp.float32),
        pltpu.SemaphoreType.DMA,
        pltpu.SemaphoreType.DMA,
    ],
)
def _sc_layer2(h1a_hbm, h1b_hbm, src_hbm, dst_hbm, zrows,
               psum, acc, idx_s, idx_d, rows0, rows1,
               sem_g0, sem_g1):
    c = lax.axis_index("c")
    s = lax.axis_index("s")

    _zero_acc(zrows, acc, s)

    rb = s * ROWS_L2
    plsc.subcore_barrier()

    def make_chunk(h_hbm):
        def chunk(k, carry):
            pltpu.sync_copy(src_hbm.at[pl.ds(rb + k * ICH, ICH)], idx_s)
            pltpu.sync_copy(dst_hbm.at[pl.ds(rb + k * ICH, ICH)], idx_d)
            pltpu.async_copy(h_hbm.at[idx_s.at[0]], rows0, sem_g0)

            def pair(p, carry2):
                j = 2 * p
                pltpu.async_copy(h_hbm.at[idx_s.at[j + 1]], rows1, sem_g1)
                pltpu.make_async_copy(h_hbm.at[idx_s.at[0]], rows0, sem_g0).wait()
                pltpu.sync_copy(rows0, acc.at[idx_d.at[j]], add=True)

                @pl.when(j + 2 < ICH)
                def _():
                    pltpu.async_copy(h_hbm.at[idx_s.at[j + 2]], rows0, sem_g0)

                pltpu.make_async_copy(h_hbm.at[idx_s.at[0]], rows1, sem_g1).wait()
                pltpu.sync_copy(rows1, acc.at[idx_d.at[j + 1]], add=True)
                return carry2

            return lax.fori_loop(0, ICH // 2, pair, carry)

        return chunk

    @pl.when(c == 0)
    def _():
        lax.fori_loop(0, ROWS_L2 // ICH, make_chunk(h1a_hbm), 0)

    @pl.when(c == 1)
    def _():
        lax.fori_loop(0, ROWS_L2 // ICH, make_chunk(h1b_hbm), 0)

    plsc.subcore_barrier()
    _copy_out_rows(acc, psum, c, s)


def _tc1_body(psum_ref, pd0_ref, pd1_ref, x_ref, wl_ref, wr_ref, b_ref,
              h1a_ref, h1b_ref, inv_ref):
    deg = pd0_ref[...] + pd1_ref[...]
    inv = 1.0 / jnp.maximum(deg, 1.0)
    aggr = (psum_ref[0] + psum_ref[1]) * inv
    h1 = (jnp.dot(aggr, wl_ref[...], preferred_element_type=jnp.float32)
          + jnp.dot(x_ref[...], wr_ref[...], preferred_element_type=jnp.float32)
          + b_ref[...])
    h1a_ref[...] = h1[:, :D_IN]
    h1b_ref[...] = h1[:, D_IN:]
    inv_ref[...] = inv


def _tc2_body(s2_ref, inv_ref, h1a_ref, h1b_ref,
              wla_ref, wlb_ref, wra_ref, wrb_ref, b_ref, out_ref):
    inv = inv_ref[...]
    o = (jnp.dot(s2_ref[0] * inv, wla_ref[...], preferred_element_type=jnp.float32)
         + jnp.dot(s2_ref[1] * inv, wlb_ref[...], preferred_element_type=jnp.float32)
         + jnp.dot(h1a_ref[...], wra_ref[...], preferred_element_type=jnp.float32)
         + jnp.dot(h1b_ref[...], wrb_ref[...], preferred_element_type=jnp.float32)
         + b_ref[...])
    out_ref[...] = jnp.maximum(o, 0.0)


_R = 2000


def _tc_layer1(psum, pd0, pd1, x, w1lT, w1rT, b1r):
    H = D_HID // 2
    return pl.pallas_call(
        _tc1_body,
        grid=(N // _R,),
        in_specs=[
            pl.BlockSpec((NC, _R, D_IN), lambda i: (0, i, 0)),
            pl.BlockSpec((_R, 1), lambda i: (i, 0)),
            pl.BlockSpec((_R, 1), lambda i: (i, 0)),
            pl.BlockSpec((_R, D_IN), lambda i: (i, 0)),
            pl.BlockSpec((D_IN, D_HID), lambda i: (0, 0)),
            pl.BlockSpec((D_IN, D_HID), lambda i: (0, 0)),
            pl.BlockSpec((1, D_HID), lambda i: (0, 0)),
        ],
        out_specs=[
            pl.BlockSpec((_R, H), lambda i: (i, 0)),
            pl.BlockSpec((_R, H), lambda i: (i, 0)),
            pl.BlockSpec((_R, 1), lambda i: (i, 0)),
        ],
        out_shape=[
            jax.ShapeDtypeStruct((N, H), jnp.float32),
            jax.ShapeDtypeStruct((N, H), jnp.float32),
            jax.ShapeDtypeStruct((N, 1), jnp.float32),
        ],
    )(psum, pd0, pd1, x, w1lT, w1rT, b1r)


def _tc_layer2(psum2, inv, h1a, h1b, w2la, w2lb, w2ra, w2rb, b2r):
    H = D_HID // 2
    return pl.pallas_call(
        _tc2_body,
        grid=(N // _R,),
        in_specs=[
            pl.BlockSpec((NC, _R, H), lambda i: (0, i, 0)),
            pl.BlockSpec((_R, 1), lambda i: (i, 0)),
            pl.BlockSpec((_R, H), lambda i: (i, 0)),
            pl.BlockSpec((_R, H), lambda i: (i, 0)),
            pl.BlockSpec((H, D_HID), lambda i: (0, 0)),
            pl.BlockSpec((H, D_HID), lambda i: (0, 0)),
            pl.BlockSpec((H, D_HID), lambda i: (0, 0)),
            pl.BlockSpec((H, D_HID), lambda i: (0, 0)),
            pl.BlockSpec((1, D_HID), lambda i: (0, 0)),
        ],
        out_specs=pl.BlockSpec((_R, D_HID), lambda i: (i, 0)),
        out_shape=jax.ShapeDtypeStruct((N, D_HID), jnp.float32),
    )(psum2, inv, h1a, h1b, w2la, w2lb, w2ra, w2rb, b2r)


def kernel(x, edge_index, W1_l, b1, W1_r, W2_l, b2, W2_r):
    ei = edge_index.astype(jnp.int32)
    pad = E_PAD - E
    src2d = jnp.concatenate([ei[0], jnp.zeros((pad,), jnp.int32)]).reshape(E_PAD // B, B)
    dst2d = jnp.concatenate([ei[1], jnp.full((pad,), N, jnp.int32)]).reshape(E_PAD // B, B)
    zrows = np.zeros((NPAD, D_IN), np.float32)
    zdeg = np.zeros((NDEG,), np.float32)

    psum, pdeg0, pdeg1 = _sc_layer1(x, src2d, dst2d, zrows, zdeg)
    h1a, h1b, inv = _tc_layer1(
        psum, pdeg0[:N].reshape(N, 1), pdeg1[:N].reshape(N, 1), x,
        W1_l.T, W1_r.T, b1.reshape(1, D_HID))
    psum2 = _sc_layer2(h1a, h1b, src2d, dst2d, zrows)
    out = _tc_layer2(
        psum2, inv, h1a, h1b,
        W2_l.T[:D_IN], W2_l.T[D_IN:], W2_r.T[:D_IN], W2_r.T[D_IN:],
        b2.reshape(1, D_HID))
    return out
